# baseline (speedup 1.0000x reference)
import jax
import jax.numpy as jnp
from jax import lax
from jax.experimental import pallas as pl
from jax.experimental.pallas import tpu as pltpu
from jax.experimental.pallas import tpu_sc as plsc

BLOCK = 16
VOCAB = 1024
D = 768

_info = plsc.get_sparse_core_info()
NC, NS, L = _info.num_cores, _info.num_subcores, _info.num_lanes
NW = NC * NS

PAIR = 2 * BLOCK


def _sc_body(tokens_hbm, cat_w_hbm, num_w_hbm, token_w_hbm,
             seq_hbm, cat_ids_hbm, hist_hbm,
             tok_v, numrows0, numrows1,
             rows_c, acc0, acc1, hist0, hist1, cat_v,
             gn0, gn1, on0, on1, oh0, oh1, sem_c, sem_d):
    numrows = (numrows0, numrows1)
    acc2 = (acc0, acc1)
    hist2 = (hist0, hist1)
    gn = (gn0, gn1)
    on = (on0, on1)
    oh = (oh0, oh1)

    n_tok = tokens_hbm.shape[0]
    blocks_total = n_tok // BLOCK
    blk_per_w = blocks_total // NW
    n_iters = blk_per_w // 2
    npb = 256
    spb = 2 * npb + 4096

    wid = lax.axis_index("s") * NC + lax.axis_index("c")
    blk0 = wid * blk_per_w
    b = blk0 // npb
    n0 = blk0 - b * npb

    zeros16 = jnp.zeros((L,), jnp.float32)
    ones16 = jnp.ones((L,), jnp.float32)
    lane = lax.iota(jnp.int32, L)

    pltpu.sync_copy(tokens_hbm.at[pl.ds(blk0 * BLOCK, blk_per_w * BLOCK)],
                    tok_v)

    for h in range(blk_per_w // L):
        cat_idx = jnp.zeros((L,), jnp.int32)
        for k in range(L):
            t0 = tok_v[pl.ds((h * L + k) * BLOCK, L)][0]
            cat_idx = jnp.where(lane == k, t0, cat_idx)
        cat_v[pl.ds(h * L, L)] = cat_idx
        pltpu.async_copy(cat_w_hbm.at[cat_idx], rows_c, sem_c).wait()
        row0 = b * spb + n0 + h * L
        pltpu.sync_copy(rows_c, seq_hbm.at[pl.ds(row0, L)])
    pltpu.sync_copy(cat_v, cat_ids_hbm.at[pl.ds(wid * blk_per_w, blk_per_w)])

    for p in (0, 1):
        for q in (0, 1):
            for i in range(VOCAB // L):
                hist2[p][q, pl.ds(i * L, L)] = zeros16

    def issue_gathers(k, p):
        idx = tok_v.at[pl.ds(k * PAIR, PAIR)]
        pltpu.async_copy(num_w_hbm.at[idx], numrows[p], gn[p])

    def wait_gathers(p):
        pltpu.make_async_copy(num_w_hbm.at[pl.ds(0, PAIR)],
                              numrows[p], gn[p]).wait()

    def drain_outs(p):
        pltpu.make_async_copy(acc2[p], seq_hbm.at[pl.ds(0, 2)], on[p]).wait()
        pltpu.make_async_copy(hist2[p], hist_hbm.at[pl.ds(0, 2)], oh[p]).wait()

    issue_gathers(0, 0)

    def outer(kk, _):
        for p in (0, 1):
            k = kk * 2 + p
            nk = k + 1
            @pl.when(jnp.logical_and(k >= 1, nk < n_iters))
            def _():
                drain_outs(1 - p)

            @pl.when(nk < n_iters)
            def _():
                issue_gathers(nk, 1 - p)

            wait_gathers(p)

            tok_row0 = b * spb + 2 * npb + (n0 + k * 2) * BLOCK
            for q in (0, 1):
                tok_idx_q = tok_v[pl.ds((k * 2 + q) * BLOCK, BLOCK)]
                for r in range(BLOCK):
                    t = tok_idx_q[r]
                    pltpu.async_copy(token_w_hbm.at[t],
                                     seq_hbm.at[tok_row0 + q * BLOCK + r],
                                     sem_d)

            @pl.when(k >= 2)
            def _():
                for q in (0, 1):
                    old_idx = tok_v[pl.ds(((k - 2) * 2 + q) * BLOCK, BLOCK)]
                    qv = jnp.full((L,), q, jnp.int32)
                    plsc.store_scatter(hist2[p], [qv, old_idx], zeros16)

            for q in (0, 1):
                tok_idx = tok_v[pl.ds((k * 2 + q) * BLOCK, BLOCK)]
                qv = jnp.full((L,), q, jnp.int32)
                plsc.addupdate_scatter(hist2[p], [qv, tok_idx], ones16)

                def chunk_body(c, _, q=q, p=p):
                    s = numrows[p][q * BLOCK, pl.ds(c * L, L)]
                    for r in range(1, BLOCK):
                        s = s + numrows[p][q * BLOCK + r, pl.ds(c * L, L)]
                    acc2[p][q, pl.ds(c * L, L)] = s
                    return 0

                lax.fori_loop(0, D // L, chunk_body, 0)

            num_row0 = b * spb + npb + n0 + k * 2
            pltpu.async_copy(acc2[p], seq_hbm.at[pl.ds(num_row0, 2)], on[p])
            pltpu.async_copy(hist2[p], hist_hbm.at[pl.ds(blk0 + k * 2, 2)],
                             oh[p])
        return 0

    lax.fori_loop(0, n_iters // 2, outer, 0)

    drain_outs(0)
    drain_outs(1)

    def tok_drain(i, _):
        pltpu.make_async_copy(token_w_hbm.at[0], seq_hbm.at[0], sem_d).wait()
        return 0

    lax.fori_loop(0, blk_per_w * BLOCK, tok_drain, 0)


def kernel(tokens, cat_W, num_W, token_W):
    B, Lseq = tokens.shape
    n_blocks = Lseq // BLOCK
    seq_rows = 2 * n_blocks + Lseq

    mesh = plsc.VectorSubcoreMesh(core_axis_name="c", subcore_axis_name="s")
    sc = pl.kernel(
        _sc_body,
        out_type=[
            jax.ShapeDtypeStruct((B * seq_rows, D), jnp.float32),
            jax.ShapeDtypeStruct((B * n_blocks,), jnp.int32),
            jax.ShapeDtypeStruct((B * n_blocks, VOCAB), jnp.float32),
        ],
        mesh=mesh,
        compiler_params=pltpu.CompilerParams(needs_layout_passes=False),
        scratch_types=[
            pltpu.VMEM((Lseq * B // NW,), jnp.int32),
            pltpu.VMEM((PAIR, D), jnp.float32),
            pltpu.VMEM((PAIR, D), jnp.float32),
            pltpu.VMEM((L, D), jnp.float32),
            pltpu.VMEM((2, D), jnp.float32),
            pltpu.VMEM((2, D), jnp.float32),
            pltpu.VMEM((2, VOCAB), jnp.float32),
            pltpu.VMEM((2, VOCAB), jnp.float32),
            pltpu.VMEM((B * n_blocks // NW,), jnp.int32),
        ] + [pltpu.SemaphoreType.DMA] * 8,
    )
    seq_flat, cat_ids_flat, hist_flat = sc(
        tokens.reshape(-1), cat_W, num_W, token_W)
    new_seq = seq_flat.reshape(B, seq_rows, D)
    cat_ids = cat_ids_flat.reshape(B, n_blocks)
    hist = hist_flat.reshape(B, n_blocks, VOCAB)
    return (new_seq, cat_ids, hist)


# speedup vs baseline: 21.8551x; 21.8551x over previous
import jax
import jax.numpy as jnp
from jax import lax
from jax.experimental import pallas as pl
from jax.experimental.pallas import tpu as pltpu
from jax.experimental.pallas import tpu_sc as plsc

BLOCK = 16
VOCAB = 1024
D = 768

_info = plsc.get_sparse_core_info()
NC, NS, L = _info.num_cores, _info.num_subcores, _info.num_lanes
NW = NC * NS

CH = 64


def _sc_body(tokens_hbm, cat_w_hbm, token_w_hbm,
             seq_hbm, cat_ids_hbm,
             tok_v, tokrows0, tokrows1, rows_c, cat_v,
             gt0, gt1, ot0, ot1, sem_c):
    tokrows = (tokrows0, tokrows1)
    gt = (gt0, gt1)
    ot = (ot0, ot1)

    n_tok = tokens_hbm.shape[0]
    blocks_total = n_tok // BLOCK
    blk_per_w = blocks_total // NW
    tok_per_w = blk_per_w * BLOCK
    n_ch = tok_per_w // CH
    npb = 256
    spb = 2 * npb + 4096

    wid = lax.axis_index("s") * NC + lax.axis_index("c")
    blk0 = wid * blk_per_w
    b = blk0 // npb
    n0 = blk0 - b * npb

    lane = lax.iota(jnp.int32, L)

    pltpu.sync_copy(tokens_hbm.at[pl.ds(blk0 * BLOCK, tok_per_w)], tok_v)

    for h in range(blk_per_w // L):
        cat_idx = jnp.zeros((L,), jnp.int32)
        for k in range(L):
            t0 = tok_v[pl.ds((h * L + k) * BLOCK, L)][0]
            cat_idx = jnp.where(lane == k, t0, cat_idx)
        cat_v[pl.ds(h * L, L)] = cat_idx
        pltpu.async_copy(cat_w_hbm.at[cat_idx], rows_c, sem_c).wait()
        row0 = b * spb + n0 + h * L
        pltpu.sync_copy(rows_c, seq_hbm.at[pl.ds(row0, L)])
    pltpu.sync_copy(cat_v, cat_ids_hbm.at[pl.ds(wid * blk_per_w, blk_per_w)])

    def issue(c, p):
        idx = tok_v.at[pl.ds(c * CH, CH)]
        pltpu.async_copy(token_w_hbm.at[idx], tokrows[p], gt[p])

    def wait_gather(p):
        pltpu.make_async_copy(token_w_hbm.at[pl.ds(0, CH)],
                              tokrows[p], gt[p]).wait()

    def drain_out(p):
        pltpu.make_async_copy(tokrows[p], seq_hbm.at[pl.ds(0, CH)],
                              ot[p]).wait()

    issue(0, 0)

    def outer(kk, _):
        for p in (0, 1):
            c = kk * 2 + p
            @pl.when(jnp.logical_and(c >= 1, c + 1 < n_ch))
            def _():
                drain_out(1 - p)

            @pl.when(c + 1 < n_ch)
            def _():
                issue(c + 1, 1 - p)

            wait_gather(p)
            row0 = b * spb + 2 * npb + (n0 + c * (CH // BLOCK)) * BLOCK
            pltpu.async_copy(tokrows[p], seq_hbm.at[pl.ds(row0, CH)], ot[p])
        return 0

    lax.fori_loop(0, n_ch // 2, outer, 0)
    drain_out(0)
    drain_out(1)


def _tc_body(tokens_ref, num_w_ref, hist_ref, num_ref):
    m = tokens_ref.shape[0]
    iota_v = lax.broadcasted_iota(jnp.int32, (m, VOCAB), 1)
    h = jnp.zeros((m, VOCAB), jnp.float32)
    for r in range(BLOCK):
        col = tokens_ref[:, r:r + 1]
        h = h + (col == iota_v).astype(jnp.float32)
    hist_ref[...] = h
    num_ref[...] = jnp.dot(h, num_w_ref[...],
                           preferred_element_type=jnp.float32)


def kernel(tokens, cat_W, num_W, token_W):
    B, Lseq = tokens.shape
    n_blocks = Lseq // BLOCK
    seq_rows = 2 * n_blocks + Lseq

    mesh = plsc.VectorSubcoreMesh(core_axis_name="c", subcore_axis_name="s")
    sc = pl.kernel(
        _sc_body,
        out_type=[
            jax.ShapeDtypeStruct((B * seq_rows, D), jnp.float32),
            jax.ShapeDtypeStruct((B * n_blocks,), jnp.int32),
        ],
        mesh=mesh,
        compiler_params=pltpu.CompilerParams(needs_layout_passes=False),
        scratch_types=[
            pltpu.VMEM((Lseq * B // NW,), jnp.int32),
            pltpu.VMEM((CH, D), jnp.float32),
            pltpu.VMEM((CH, D), jnp.float32),
            pltpu.VMEM((L, D), jnp.float32),
            pltpu.VMEM((B * n_blocks // NW,), jnp.int32),
        ] + [pltpu.SemaphoreType.DMA] * 5,
    )

    tc = pl.pallas_call(
        _tc_body,
        out_shape=[
            jax.ShapeDtypeStruct((B * n_blocks, VOCAB), jnp.float32),
            jax.ShapeDtypeStruct((B * n_blocks, D), jnp.float32),
        ],
    )

    seq_flat, cat_ids_flat = sc(tokens.reshape(-1), cat_W, token_W)
    hist_flat, num_flat = tc(tokens.reshape(B * n_blocks, BLOCK), num_W)

    new_seq = seq_flat.reshape(B, seq_rows, D)
    new_seq = lax.dynamic_update_slice(
        new_seq, num_flat.reshape(B, n_blocks, D), (0, n_blocks, 0))
    cat_ids = cat_ids_flat.reshape(B, n_blocks)
    hist = hist_flat.reshape(B, n_blocks, VOCAB)
    return (new_seq, cat_ids, hist)


# speedup vs baseline: 22.3848x; 1.0242x over previous
import jax
import jax.numpy as jnp
from jax import lax
from jax.experimental import pallas as pl
from jax.experimental.pallas import tpu as pltpu
from jax.experimental.pallas import tpu_sc as plsc

BLOCK = 16
VOCAB = 1024
D = 768

_info = plsc.get_sparse_core_info()
NC, NS, L = _info.num_cores, _info.num_subcores, _info.num_lanes
NW = NC * NS

CH = 64


def _sc_body(tokens_hbm, token_w_hbm, seq_hbm,
             tok_v, tokrows0, tokrows1, gt0, gt1, ot0, ot1):
    tokrows = (tokrows0, tokrows1)
    gt = (gt0, gt1)
    ot = (ot0, ot1)

    batch, lseq = tokens_hbm.shape
    blocks_total = batch * lseq // BLOCK
    blk_per_w = blocks_total // NW
    tok_per_w = blk_per_w * BLOCK
    n_ch = tok_per_w // CH
    npb = lseq // BLOCK
    spb = 2 * npb + lseq
    w_per_b = NW // batch

    wid = lax.axis_index("s") * NC + lax.axis_index("c")
    b = wid // w_per_b
    col0 = (wid - b * w_per_b) * tok_per_w

    pltpu.sync_copy(tokens_hbm.at[b, pl.ds(col0, tok_per_w)], tok_v)

    def issue(c, p):
        idx = tok_v.at[pl.ds(c * CH, CH)]
        pltpu.async_copy(token_w_hbm.at[idx], tokrows[p], gt[p])

    def wait_gather(p):
        pltpu.make_async_copy(token_w_hbm.at[pl.ds(0, CH)],
                              tokrows[p], gt[p]).wait()

    def drain_out(p):
        pltpu.make_async_copy(tokrows[p], seq_hbm.at[pl.ds(0, CH)],
                              ot[p]).wait()

    issue(0, 0)

    def outer(kk, _):
        for p in (0, 1):
            c = kk * 2 + p
            @pl.when(jnp.logical_and(c >= 1, c + 1 < n_ch))
            def _():
                drain_out(1 - p)

            @pl.when(c + 1 < n_ch)
            def _():
                issue(c + 1, 1 - p)

            wait_gather(p)
            row0 = b * spb + 2 * npb + col0 + c * CH
            pltpu.async_copy(tokrows[p], seq_hbm.at[pl.ds(row0, CH)], ot[p])
        return 0

    lax.fori_loop(0, n_ch // 2, outer, 0)
    drain_out(0)
    drain_out(1)


def _tc_body(tokens_ref, cat_w_ref, num_w_ref,
             hist_ref, catnum_ref, cat_ids_ref):
    m = tokens_ref.shape[0]
    npb = 256
    batch = m // npb
    iota_v = lax.broadcasted_iota(jnp.int32, (m, VOCAB), 1)

    col0 = tokens_ref[:, 0:1]
    cat_ids_ref[...] = col0

    h = jnp.zeros((m, VOCAB), jnp.float32)
    for r in range(BLOCK):
        col = tokens_ref[:, r:r + 1]
        h = h + (col == iota_v).astype(jnp.float32)
    hist_ref[...] = h

    onehot = (col0 == iota_v).astype(jnp.float32)
    cat = jnp.dot(onehot, cat_w_ref[...], preferred_element_type=jnp.float32)
    num = jnp.dot(h, num_w_ref[...], preferred_element_type=jnp.float32)
    for b in range(batch):
        catnum_ref[b * 2 * npb:b * 2 * npb + npb, :] = (
            cat[b * npb:(b + 1) * npb, :])
        catnum_ref[b * 2 * npb + npb:(b + 1) * 2 * npb, :] = (
            num[b * npb:(b + 1) * npb, :])


def kernel(tokens, cat_W, num_W, token_W):
    B, Lseq = tokens.shape
    n_blocks = Lseq // BLOCK
    seq_rows = 2 * n_blocks + Lseq

    mesh = plsc.VectorSubcoreMesh(core_axis_name="c", subcore_axis_name="s")
    sc = pl.kernel(
        _sc_body,
        out_type=jax.ShapeDtypeStruct((B * seq_rows, D), jnp.float32),
        mesh=mesh,
        compiler_params=pltpu.CompilerParams(needs_layout_passes=False),
        scratch_types=[
            pltpu.VMEM((Lseq * B // NW,), jnp.int32),
            pltpu.VMEM((CH, D), jnp.float32),
            pltpu.VMEM((CH, D), jnp.float32),
        ] + [pltpu.SemaphoreType.DMA] * 4,
    )

    tc = pl.pallas_call(
        _tc_body,
        out_shape=[
            jax.ShapeDtypeStruct((B * n_blocks, VOCAB), jnp.float32),
            jax.ShapeDtypeStruct((B * 2 * n_blocks, D), jnp.float32),
            jax.ShapeDtypeStruct((B * n_blocks, 1), jnp.int32),
        ],
    )

    seq_flat = sc(tokens, token_W)
    hist_flat, catnum, cat_ids_col = tc(
        tokens.reshape(B * n_blocks, BLOCK), cat_W, num_W)

    new_seq = seq_flat.reshape(B, seq_rows, D)
    new_seq = lax.dynamic_update_slice(
        new_seq, catnum.reshape(B, 2 * n_blocks, D), (0, 0, 0))
    cat_ids = cat_ids_col.reshape(B, n_blocks)
    hist = hist_flat.reshape(B, n_blocks, VOCAB)
    return (new_seq, cat_ids, hist)


# speedup vs baseline: 22.5495x; 1.0074x over previous
import jax
import jax.numpy as jnp
from jax import lax
from jax.experimental import pallas as pl
from jax.experimental.pallas import tpu as pltpu
from jax.experimental.pallas import tpu_sc as plsc

BLOCK = 16
VOCAB = 1024
D = 768

_info = plsc.get_sparse_core_info()
NC, NS, L = _info.num_cores, _info.num_subcores, _info.num_lanes
NW = NC * NS

CH = 32
RING = 4
LOOK = 2


def _sc_body(tokens_hbm, token_w_hbm, seq_hbm,
             tok_v, tokrows0, tokrows1, tokrows2, tokrows3,
             gt0, gt1, gt2, gt3, ot0, ot1, ot2, ot3):
    tokrows = (tokrows0, tokrows1, tokrows2, tokrows3)
    gt = (gt0, gt1, gt2, gt3)
    ot = (ot0, ot1, ot2, ot3)

    batch, lseq = tokens_hbm.shape
    blocks_total = batch * lseq // BLOCK
    blk_per_w = blocks_total // NW
    tok_per_w = blk_per_w * BLOCK
    n_ch = tok_per_w // CH
    npb = lseq // BLOCK
    spb = 2 * npb + lseq
    w_per_b = NW // batch

    wid = lax.axis_index("s") * NC + lax.axis_index("c")
    b = wid // w_per_b
    col0 = (wid - b * w_per_b) * tok_per_w

    pltpu.sync_copy(tokens_hbm.at[b, pl.ds(col0, tok_per_w)], tok_v)

    def issue(c, p):
        idx = tok_v.at[pl.ds(c * CH, CH)]
        pltpu.async_copy(token_w_hbm.at[idx], tokrows[p], gt[p])

    def wait_gather(p):
        pltpu.make_async_copy(token_w_hbm.at[pl.ds(0, CH)],
                              tokrows[p], gt[p]).wait()

    def drain_out(p):
        pltpu.make_async_copy(tokrows[p], seq_hbm.at[pl.ds(0, CH)],
                              ot[p]).wait()

    for c0 in range(LOOK):
        issue(c0, c0)

    def outer(kk, _):
        for i in range(RING):
            c = kk * RING + i
            nc = c + LOOK
            nbuf = (i + LOOK) % RING

            @pl.when(jnp.logical_and(c >= RING - LOOK, nc < n_ch))
            def _():
                drain_out(nbuf)

            @pl.when(nc < n_ch)
            def _():
                issue(nc, nbuf)

            wait_gather(i)
            row0 = b * spb + 2 * npb + col0 + c * CH
            pltpu.async_copy(tokrows[i], seq_hbm.at[pl.ds(row0, CH)], ot[i])
        return 0

    lax.fori_loop(0, n_ch // RING, outer, 0)
    for i in range(RING):
        drain_out(i)


def _tc_body(tokens_ref, cat_w_ref, num_w_ref,
             hist_ref, catnum_ref, cat_ids_ref):
    m = tokens_ref.shape[0]
    npb = 256
    batch = m // npb
    iota_v = lax.broadcasted_iota(jnp.int32, (m, VOCAB), 1)

    col0 = tokens_ref[:, 0:1]
    cat_ids_ref[...] = col0

    h = jnp.zeros((m, VOCAB), jnp.float32)
    for r in range(BLOCK):
        col = tokens_ref[:, r:r + 1]
        h = h + (col == iota_v).astype(jnp.float32)
    hist_ref[...] = h

    onehot = (col0 == iota_v).astype(jnp.float32)
    cat = jnp.dot(onehot, cat_w_ref[...], preferred_element_type=jnp.float32)
    num = jnp.dot(h, num_w_ref[...], preferred_element_type=jnp.float32)
    for b in range(batch):
        catnum_ref[b * 2 * npb:b * 2 * npb + npb, :] = (
            cat[b * npb:(b + 1) * npb, :])
        catnum_ref[b * 2 * npb + npb:(b + 1) * 2 * npb, :] = (
            num[b * npb:(b + 1) * npb, :])


def kernel(tokens, cat_W, num_W, token_W):
    B, Lseq = tokens.shape
    n_blocks = Lseq // BLOCK
    seq_rows = 2 * n_blocks + Lseq

    mesh = plsc.VectorSubcoreMesh(core_axis_name="c", subcore_axis_name="s")
    sc = pl.kernel(
        _sc_body,
        out_type=jax.ShapeDtypeStruct((B * seq_rows, D), jnp.float32),
        mesh=mesh,
        compiler_params=pltpu.CompilerParams(needs_layout_passes=False),
        scratch_types=[
            pltpu.VMEM((Lseq * B // NW,), jnp.int32),
        ] + [pltpu.VMEM((CH, D), jnp.float32)] * RING
          + [pltpu.SemaphoreType.DMA] * (2 * RING),
    )

    tc = pl.pallas_call(
        _tc_body,
        out_shape=[
            jax.ShapeDtypeStruct((B * n_blocks, VOCAB), jnp.float32),
            jax.ShapeDtypeStruct((B * 2 * n_blocks, D), jnp.float32),
            jax.ShapeDtypeStruct((B * n_blocks, 1), jnp.int32),
        ],
    )

    seq_flat = sc(tokens, token_W)
    hist_flat, catnum, cat_ids_col = tc(
        tokens.reshape(B * n_blocks, BLOCK), cat_W, num_W)

    new_seq = seq_flat.reshape(B, seq_rows, D)
    new_seq = lax.dynamic_update_slice(
        new_seq, catnum.reshape(B, 2 * n_blocks, D), (0, 0, 0))
    cat_ids = cat_ids_col.reshape(B, n_blocks)
    hist = hist_flat.reshape(B, n_blocks, VOCAB)
    return (new_seq, cat_ids, hist)


# speedup vs baseline: 22.7348x; 1.0082x over previous
import jax
import jax.numpy as jnp
from jax import lax
from jax.experimental import pallas as pl
from jax.experimental.pallas import tpu as pltpu
from jax.experimental.pallas import tpu_sc as plsc

BLOCK = 16
VOCAB = 1024
D = 768

_info = plsc.get_sparse_core_info()
NC, NS, L = _info.num_cores, _info.num_subcores, _info.num_lanes
NW = NC * NS

CH = 32
RING = 4
LOOK = 2


def _sc_body(tokens_hbm, token_w_hbm, seq_hbm,
             tok_v, tokrows0, tokrows1, tokrows2, tokrows3,
             gt0, gt1, gt2, gt3, ot0, ot1, ot2, ot3):
    tokrows = (tokrows0, tokrows1, tokrows2, tokrows3)
    gt = (gt0, gt1, gt2, gt3)
    ot = (ot0, ot1, ot2, ot3)

    batch, lseq = tokens_hbm.shape
    blocks_total = batch * lseq // BLOCK
    blk_per_w = blocks_total // NW
    tok_per_w = blk_per_w * BLOCK
    n_ch = tok_per_w // CH
    npb = lseq // BLOCK
    spb = 2 * npb + lseq
    w_per_b = NW // batch

    wid = lax.axis_index("s") * NC + lax.axis_index("c")
    b = wid // w_per_b
    col0 = (wid - b * w_per_b) * tok_per_w

    pltpu.sync_copy(tokens_hbm.at[b, pl.ds(col0, tok_per_w)], tok_v)

    def issue(c, p):
        idx = tok_v.at[pl.ds(c * CH, CH)]
        pltpu.async_copy(token_w_hbm.at[idx], tokrows[p], gt[p])

    def wait_gather(p):
        pltpu.make_async_copy(token_w_hbm.at[pl.ds(0, CH)],
                              tokrows[p], gt[p]).wait()

    def drain_out(p):
        pltpu.make_async_copy(tokrows[p], seq_hbm.at[pl.ds(0, CH)],
                              ot[p]).wait()

    for c0 in range(LOOK):
        issue(c0, c0)

    def outer(kk, _):
        for i in range(RING):
            c = kk * RING + i
            nc = c + LOOK
            nbuf = (i + LOOK) % RING

            @pl.when(jnp.logical_and(c >= RING - LOOK, nc < n_ch))
            def _():
                drain_out(nbuf)

            @pl.when(nc < n_ch)
            def _():
                issue(nc, nbuf)

            wait_gather(i)
            row0 = b * spb + 2 * npb + col0 + c * CH
            pltpu.async_copy(tokrows[i], seq_hbm.at[pl.ds(row0, CH)], ot[i])
        return 0

    lax.fori_loop(0, n_ch // RING, outer, 0)
    for i in range(RING):
        drain_out(i)


def _tc_body(tokens_ref, cat_w_ref, num_w_ref,
             hist_ref, catnum_ref, cat_ids_ref):
    m = tokens_ref.shape[0]
    npb = 256
    batch = m // npb
    iota_v = lax.broadcasted_iota(jnp.int32, (m, VOCAB), 1)

    col0 = tokens_ref[:, 0:1]
    cat_ids_ref[...] = col0.reshape(batch, npb)

    h = jnp.zeros((m, VOCAB), jnp.float32)
    for r in range(BLOCK):
        col = tokens_ref[:, r:r + 1]
        h = h + (col == iota_v).astype(jnp.float32)
    hist_ref[...] = h

    onehot = (col0 == iota_v).astype(jnp.float32)
    cat = jnp.dot(onehot, cat_w_ref[...], preferred_element_type=jnp.float32)
    num = jnp.dot(h, num_w_ref[...], preferred_element_type=jnp.float32)
    for b in range(batch):
        catnum_ref[b * 2 * npb:b * 2 * npb + npb, :] = (
            cat[b * npb:(b + 1) * npb, :])
        catnum_ref[b * 2 * npb + npb:(b + 1) * 2 * npb, :] = (
            num[b * npb:(b + 1) * npb, :])


def kernel(tokens, cat_W, num_W, token_W):
    B, Lseq = tokens.shape
    n_blocks = Lseq // BLOCK
    seq_rows = 2 * n_blocks + Lseq

    mesh = plsc.VectorSubcoreMesh(core_axis_name="c", subcore_axis_name="s")
    sc = pl.kernel(
        _sc_body,
        out_type=jax.ShapeDtypeStruct((B * seq_rows, D), jnp.float32),
        mesh=mesh,
        compiler_params=pltpu.CompilerParams(needs_layout_passes=False),
        scratch_types=[
            pltpu.VMEM((Lseq * B // NW,), jnp.int32),
        ] + [pltpu.VMEM((CH, D), jnp.float32)] * RING
          + [pltpu.SemaphoreType.DMA] * (2 * RING),
    )

    tc = pl.pallas_call(
        _tc_body,
        out_shape=[
            jax.ShapeDtypeStruct((B * n_blocks, VOCAB), jnp.float32),
            jax.ShapeDtypeStruct((B * 2 * n_blocks, D), jnp.float32),
            jax.ShapeDtypeStruct((B, n_blocks), jnp.int32),
        ],
    )

    seq_flat = sc(tokens, token_W)
    hist_flat, catnum, cat_ids = tc(
        tokens.reshape(B * n_blocks, BLOCK), cat_W, num_W)

    new_seq = seq_flat.reshape(B, seq_rows, D)
    new_seq = lax.dynamic_update_slice(
        new_seq, catnum.reshape(B, 2 * n_blocks, D), (0, 0, 0))
    hist = hist_flat.reshape(B, n_blocks, VOCAB)
    return (new_seq, cat_ids, hist)


# speedup vs baseline: 22.9907x; 1.0113x over previous
import jax
import jax.numpy as jnp
from jax import lax
from jax.experimental import pallas as pl
from jax.experimental.pallas import tpu as pltpu
from jax.experimental.pallas import tpu_sc as plsc

BLOCK = 16
VOCAB = 1024
D = 768

_info = plsc.get_sparse_core_info()
NC, NS, L = _info.num_cores, _info.num_subcores, _info.num_lanes
NW = NC * NS

CH = 16
RING = 8
LOOK = 4


def _sc_body(tokens_hbm, token_w_hbm, seq_hbm,
             tok_v, tokrows0, tokrows1, tokrows2, tokrows3,
             tokrows4, tokrows5, tokrows6, tokrows7,
             gt0, gt1, gt2, gt3, gt4, gt5, gt6, gt7,
             ot0, ot1, ot2, ot3, ot4, ot5, ot6, ot7):
    tokrows = (tokrows0, tokrows1, tokrows2, tokrows3,
               tokrows4, tokrows5, tokrows6, tokrows7)
    gt = (gt0, gt1, gt2, gt3, gt4, gt5, gt6, gt7)
    ot = (ot0, ot1, ot2, ot3, ot4, ot5, ot6, ot7)

    batch, lseq = tokens_hbm.shape
    blocks_total = batch * lseq // BLOCK
    blk_per_w = blocks_total // NW
    tok_per_w = blk_per_w * BLOCK
    n_ch = tok_per_w // CH
    npb = lseq // BLOCK
    spb = 2 * npb + lseq
    w_per_b = NW // batch

    wid = lax.axis_index("s") * NC + lax.axis_index("c")
    b = wid // w_per_b
    col0 = (wid - b * w_per_b) * tok_per_w

    pltpu.sync_copy(tokens_hbm.at[b, pl.ds(col0, tok_per_w)], tok_v)

    def issue(c, p):
        idx = tok_v.at[pl.ds(c * CH, CH)]
        pltpu.async_copy(token_w_hbm.at[idx], tokrows[p], gt[p])

    def wait_gather(p):
        pltpu.make_async_copy(token_w_hbm.at[pl.ds(0, CH)],
                              tokrows[p], gt[p]).wait()

    def drain_out(p):
        pltpu.make_async_copy(tokrows[p], seq_hbm.at[pl.ds(0, CH)],
                              ot[p]).wait()

    for c0 in range(LOOK):
        issue(c0, c0)

    def outer(kk, _):
        for i in range(RING):
            c = kk * RING + i
            nc = c + LOOK
            nbuf = (i + LOOK) % RING

            @pl.when(jnp.logical_and(c >= RING - LOOK, nc < n_ch))
            def _():
                drain_out(nbuf)

            @pl.when(nc < n_ch)
            def _():
                issue(nc, nbuf)

            wait_gather(i)
            row0 = b * spb + 2 * npb + col0 + c * CH
            pltpu.async_copy(tokrows[i], seq_hbm.at[pl.ds(row0, CH)], ot[i])
        return 0

    lax.fori_loop(0, n_ch // RING, outer, 0)
    for i in range(RING):
        drain_out(i)


def _tc_body(tokens_ref, cat_w_ref, num_w_ref,
             hist_ref, catnum_ref, cat_ids_ref):
    m = tokens_ref.shape[0]
    npb = 256
    batch = m // npb
    iota_v = lax.broadcasted_iota(jnp.int32, (m, VOCAB), 1)

    col0 = tokens_ref[:, 0:1]
    cat_ids_ref[...] = col0.reshape(batch, npb)

    h = jnp.zeros((m, VOCAB), jnp.float32)
    for r in range(BLOCK):
        col = tokens_ref[:, r:r + 1]
        h = h + (col == iota_v).astype(jnp.float32)
    hist_ref[...] = h

    onehot = (col0 == iota_v).astype(jnp.float32)
    cat = jnp.dot(onehot, cat_w_ref[...], preferred_element_type=jnp.float32)
    num = jnp.dot(h, num_w_ref[...], preferred_element_type=jnp.float32)
    for b in range(batch):
        catnum_ref[b * 2 * npb:b * 2 * npb + npb, :] = (
            cat[b * npb:(b + 1) * npb, :])
        catnum_ref[b * 2 * npb + npb:(b + 1) * 2 * npb, :] = (
            num[b * npb:(b + 1) * npb, :])


def kernel(tokens, cat_W, num_W, token_W):
    B, Lseq = tokens.shape
    n_blocks = Lseq // BLOCK
    seq_rows = 2 * n_blocks + Lseq

    mesh = plsc.VectorSubcoreMesh(core_axis_name="c", subcore_axis_name="s")
    sc = pl.kernel(
        _sc_body,
        out_type=jax.ShapeDtypeStruct((B * seq_rows, D), jnp.float32),
        mesh=mesh,
        compiler_params=pltpu.CompilerParams(needs_layout_passes=False),
        scratch_types=[
            pltpu.VMEM((Lseq * B // NW,), jnp.int32),
        ] + [pltpu.VMEM((CH, D), jnp.float32)] * RING
          + [pltpu.SemaphoreType.DMA] * (2 * RING),
    )

    tc = pl.pallas_call(
        _tc_body,
        out_shape=[
            jax.ShapeDtypeStruct((B * n_blocks, VOCAB), jnp.float32),
            jax.ShapeDtypeStruct((B * 2 * n_blocks, D), jnp.float32),
            jax.ShapeDtypeStruct((B, n_blocks), jnp.int32),
        ],
    )

    seq_flat = sc(tokens, token_W)
    hist_flat, catnum, cat_ids = tc(
        tokens.reshape(B * n_blocks, BLOCK), cat_W, num_W)

    new_seq = seq_flat.reshape(B, seq_rows, D)
    new_seq = lax.dynamic_update_slice(
        new_seq, catnum.reshape(B, 2 * n_blocks, D), (0, 0, 0))
    hist = hist_flat.reshape(B, n_blocks, VOCAB)
    return (new_seq, cat_ids, hist)
